# baseline (speedup 1.0000x reference)
import functools

import jax
import jax.numpy as jnp
from jax import lax
from jax.experimental import pallas as pl
from jax.experimental.pallas import tpu as pltpu
from jax.experimental.pallas import tpu_sc as plsc

N = 10000
E = 320000
D = 128
DH = D // 2
BN_EPS = 1e-5

NC = 2
NS = 16
E_PER_T = E // NS
CHUNK = 125
NCHUNK = E_PER_T // CHUNK
NBUF = 5
ROWS_PER_TILE = 624
ROWS_LAST = N - (NS - 1) * ROWS_PER_TILE
ZR = 16


def _gelu(v):
    return v * 0.5 * (1.0 + lax.erf(v * 0.7071067811865476))


BLK = 1000
NBLK = N // BLK


def _enc_body(x_ref, w_ref, o0_ref, o1_ref):
    h = jnp.dot(x_ref[...], w_ref[...], preferred_element_type=jnp.float32)
    o0_ref[...] = h[:, :DH]
    o1_ref[...] = h[:, DH:]


def _enc_matmul(x, w):
    return pl.pallas_call(
        _enc_body,
        grid=(NBLK,),
        in_specs=[pl.BlockSpec((BLK, D), lambda i: (i, 0)),
                  pl.BlockSpec((D, D), lambda i: (0, 0))],
        out_specs=[pl.BlockSpec((BLK, DH), lambda i: (i, 0)),
                   pl.BlockSpec((BLK, DH), lambda i: (i, 0))],
        out_shape=[jax.ShapeDtypeStruct((N, DH), jnp.float32),
                   jax.ShapeDtypeStruct((N, DH), jnp.float32)],
    )(x, w)


def _mlpA_body(h0_ref, h1_ref, a0_ref, a1_ref, w1_ref, w2_ref,
               o_ref, st_ref, s1_acc, s2_acc):
    i = pl.program_id(0)
    m = jnp.concatenate(
        [h0_ref[...] + a0_ref[...], h1_ref[...] + a1_ref[...]], axis=1)
    hid = _gelu(jnp.dot(m, w1_ref[...], preferred_element_type=jnp.float32))
    o = jnp.dot(hid, w2_ref[...], preferred_element_type=jnp.float32)
    o_ref[...] = o
    ones = jnp.ones((8, BLK), jnp.float32)
    s1 = jnp.dot(ones, o, preferred_element_type=jnp.float32,
                 precision=lax.Precision.HIGHEST)
    s2 = jnp.dot(ones, o * o, preferred_element_type=jnp.float32,
                 precision=lax.Precision.HIGHEST)

    @pl.when(i == 0)
    def _():
        s1_acc[...] = s1
        s2_acc[...] = s2

    @pl.when(i > 0)
    def _():
        s1_acc[...] += s1
        s2_acc[...] += s2

    @pl.when(i == NBLK - 1)
    def _():
        st_ref[0:8] = s1_acc[...]
        st_ref[8:16] = s2_acc[...]


def _mlpB_body(apply_act, o_ref, st_ref, g_ref, b_ref, *o_refs):
    mean = st_ref[0:1] * (1.0 / N)
    var = st_ref[8:9] * (1.0 / N) - mean * mean
    inv = lax.rsqrt(var + BN_EPS)
    out = (o_ref[...] - mean) * inv * g_ref[...][None, :] + b_ref[...][None, :]
    if apply_act:
        out = _gelu(out)
        o_refs[0][...] = out[:, :DH]
        o_refs[1][...] = out[:, DH:]
    else:
        o_refs[0][...] = out


def _mlp(h0, h1, a0, a1, w1, w2, gamma, beta, apply_act):
    o, st = pl.pallas_call(
        _mlpA_body,
        grid=(NBLK,),
        in_specs=[pl.BlockSpec((BLK, DH), lambda i: (i, 0)),
                  pl.BlockSpec((BLK, DH), lambda i: (i, 0)),
                  pl.BlockSpec((BLK, DH), lambda i: (i, 0)),
                  pl.BlockSpec((BLK, DH), lambda i: (i, 0)),
                  pl.BlockSpec((D, 2 * D), lambda i: (0, 0)),
                  pl.BlockSpec((2 * D, D), lambda i: (0, 0))],
        out_specs=[pl.BlockSpec((BLK, D), lambda i: (i, 0)),
                   pl.BlockSpec((16, D), lambda i: (0, 0))],
        out_shape=[jax.ShapeDtypeStruct((N, D), jnp.float32),
                   jax.ShapeDtypeStruct((16, D), jnp.float32)],
        scratch_shapes=[pltpu.VMEM((8, D), jnp.float32),
                        pltpu.VMEM((8, D), jnp.float32)],
    )(h0, h1, a0, a1, w1, w2)
    if apply_act:
        out_specs = [pl.BlockSpec((BLK, DH), lambda i: (i, 0)),
                     pl.BlockSpec((BLK, DH), lambda i: (i, 0))]
        shapes = [jax.ShapeDtypeStruct((N, DH), jnp.float32),
                  jax.ShapeDtypeStruct((N, DH), jnp.float32)]
    else:
        out_specs = [pl.BlockSpec((BLK, D), lambda i: (i, 0))]
        shapes = [jax.ShapeDtypeStruct((N, D), jnp.float32)]
    return pl.pallas_call(
        functools.partial(_mlpB_body, apply_act),
        grid=(NBLK,),
        in_specs=[pl.BlockSpec((BLK, D), lambda i: (i, 0)),
                  pl.BlockSpec((16, D), lambda i: (0, 0)),
                  pl.BlockSpec((D,), lambda i: (0,)),
                  pl.BlockSpec((D,), lambda i: (0,))],
        out_specs=out_specs,
        out_shape=shapes,
    )(o, st, gamma, beta)


def _seg_body(h0_hbm, h1_hbm, edge_hbm, out0_hbm, out1_hbm, *scr):
    sidx_v, didx_v = scr[0], scr[1]
    bufs = scr[2:2 + NBUF]
    acc_sh = scr[2 + NBUF]
    gsems = scr[3 + NBUF:3 + 2 * NBUF]
    ssems = scr[3 + 2 * NBUF:3 + 3 * NBUF]
    buf0 = bufs[0]
    c = lax.axis_index("c")
    s = lax.axis_index("s")

    pltpu.async_copy(edge_hbm.at[0, s], sidx_v, gsems[1])
    pltpu.async_copy(edge_hbm.at[1, s], didx_v, gsems[2])

    def zrow(i, _):
        def zcol(j, _):
            buf0[i, pl.ds(j * 16, 16)] = jnp.zeros((16,), jnp.float32)
            return 0
        return lax.fori_loop(0, DH // 16, zcol, 0)
    lax.fori_loop(0, ZR, zrow, 0)
    base = s * ROWS_PER_TILE
    nzk = jnp.where(s == NS - 1, ROWS_LAST // ZR, ROWS_PER_TILE // ZR)
    def zcopy(k, _):
        pltpu.async_copy(buf0.at[pl.ds(0, ZR)],
                         acc_sh.at[pl.ds(base + k * ZR, ZR)], gsems[0])
        return 0
    lax.fori_loop(0, nzk, zcopy, 0)
    def zdrain(k, _):
        pltpu.make_async_copy(buf0.at[pl.ds(0, ZR)],
                              acc_sh.at[pl.ds(base, ZR)], gsems[0]).wait()
        return 0
    lax.fori_loop(0, nzk, zdrain, 0)
    pltpu.make_async_copy(edge_hbm.at[0, s], sidx_v, gsems[1]).wait()
    pltpu.make_async_copy(edge_hbm.at[1, s], didx_v, gsems[2]).wait()
    plsc.subcore_barrier()

    def run(h_hbm):
        for b in range(NBUF):
            pltpu.async_copy(h_hbm.at[sidx_v.at[b]], bufs[b], gsems[b])

        def step(j, _):
            e0 = j * NBUF
            for b in range(NBUF):
                pltpu.make_async_copy(
                    h_hbm.at[sidx_v.at[e0 + b]], bufs[b], gsems[b]).wait()
                pltpu.async_copy(bufs[b], acc_sh.at[didx_v.at[e0 + b]],
                                 ssems[b], add=True)
            for b in range(NBUF):
                e2 = e0 + NBUF + b

                @pl.when(e2 < NCHUNK)
                def _():
                    pltpu.make_async_copy(
                        bufs[b], acc_sh.at[didx_v.at[e0 + b]],
                        ssems[b]).wait()
                    pltpu.async_copy(h_hbm.at[sidx_v.at[e2]], bufs[b],
                                     gsems[b])
            return 0
        lax.fori_loop(0, NCHUNK // NBUF, step, 0)
        for b in range(NBUF):
            pltpu.make_async_copy(
                bufs[b], acc_sh.at[didx_v.at[NCHUNK - NBUF + b]],
                ssems[b]).wait()

    @pl.when(c == 0)
    def _():
        run(h0_hbm)

    @pl.when(c == 1)
    def _():
        run(h1_hbm)

    plsc.subcore_barrier()

    def writeback(out_hbm):
        @pl.when(s < NS - 1)
        def _():
            pltpu.sync_copy(acc_sh.at[pl.ds(base, ROWS_PER_TILE)],
                            out_hbm.at[pl.ds(base, ROWS_PER_TILE)])

        @pl.when(s == NS - 1)
        def _():
            last = (NS - 1) * ROWS_PER_TILE
            pltpu.sync_copy(acc_sh.at[pl.ds(last, ROWS_LAST)],
                            out_hbm.at[pl.ds(last, ROWS_LAST)])

    @pl.when(c == 0)
    def _():
        writeback(out0_hbm)

    @pl.when(c == 1)
    def _():
        writeback(out1_hbm)


def _seg_sum(h0, h1, edge4):
    mesh = plsc.VectorSubcoreMesh(core_axis_name="c", subcore_axis_name="s")
    kern = pl.kernel(
        _seg_body,
        out_type=[jax.ShapeDtypeStruct((N, DH), jnp.float32),
                  jax.ShapeDtypeStruct((N, DH), jnp.float32)],
        mesh=mesh,
        compiler_params=pltpu.CompilerParams(use_tc_tiling_on_sc=False),
        scratch_types=(
            [pltpu.VMEM((NCHUNK, CHUNK), jnp.int32),
             pltpu.VMEM((NCHUNK, CHUNK), jnp.int32)]
            + [pltpu.VMEM((CHUNK, DH), jnp.float32)] * NBUF
            + [pltpu.VMEM_SHARED((N, DH), jnp.float32)]
            + [pltpu.SemaphoreType.DMA] * (2 * NBUF)
        ),
    )
    return kern(h0, h1, edge4)


def kernel(x, edge_index, W_enc, W1_0, W2_0, gamma_0, beta_0,
           W1_1, W2_1, gamma_1, beta_1):
    edge4 = edge_index.reshape(2, NS, NCHUNK, CHUNK)
    h0, h1 = _enc_matmul(x, W_enc)
    a0, a1 = _seg_sum(h0, h1, edge4)
    h0, h1 = _mlp(h0, h1, a0, a1, W1_0, W2_0, gamma_0, beta_0, True)
    a0, a1 = _seg_sum(h0, h1, edge4)
    (out,) = _mlp(h0, h1, a0, a1, W1_1, W2_1, gamma_1, beta_1, False)
    return out


# speedup vs baseline: 1.0385x; 1.0385x over previous
import functools

import jax
import jax.numpy as jnp
from jax import lax
from jax.experimental import pallas as pl
from jax.experimental.pallas import tpu as pltpu
from jax.experimental.pallas import tpu_sc as plsc

N = 10000
E = 320000
D = 128
DH = D // 2
BN_EPS = 1e-5

NC = 2
NS = 16
E_PER_T = E // NS
CHUNK = 125
NCHUNK = E_PER_T // CHUNK
NBUF = 5
ROWS_PER_TILE = 624
ROWS_LAST = N - (NS - 1) * ROWS_PER_TILE
ZR = 16


def _gelu(v):
    return v * 0.5 * (1.0 + lax.erf(v * 0.7071067811865476))


BLK = 1000
NBLK = N // BLK


def _enc_body(x_ref, w_ref, o0_ref, o1_ref):
    h = jnp.dot(x_ref[...], w_ref[...], preferred_element_type=jnp.float32)
    o0_ref[...] = h[:, :DH]
    o1_ref[...] = h[:, DH:]


def _enc_matmul(x, w):
    return pl.pallas_call(
        _enc_body,
        grid=(NBLK,),
        in_specs=[pl.BlockSpec((BLK, D), lambda i: (i, 0)),
                  pl.BlockSpec((D, D), lambda i: (0, 0))],
        out_specs=[pl.BlockSpec((BLK, DH), lambda i: (i, 0)),
                   pl.BlockSpec((BLK, DH), lambda i: (i, 0))],
        out_shape=[jax.ShapeDtypeStruct((N, DH), jnp.float32),
                   jax.ShapeDtypeStruct((N, DH), jnp.float32)],
    )(x, w)


def _mlp_body(apply_act, h0_ref, h1_ref, a0_ref, a1_ref,
              w1_ref, w2_ref, g_ref, b_ref, *rest):
    o_sc, s1_acc, s2_acc = rest[-3:]
    o_refs = rest[:-3]
    pid = pl.program_id(0)

    @pl.when(pid < NBLK)
    def _():
        m = jnp.concatenate(
            [h0_ref[...] + a0_ref[...], h1_ref[...] + a1_ref[...]], axis=1)
        hid = _gelu(jnp.dot(m, w1_ref[...],
                            preferred_element_type=jnp.float32))
        o = jnp.dot(hid, w2_ref[...], preferred_element_type=jnp.float32)
        off = pl.multiple_of(pid * BLK, BLK)
        o_sc[pl.ds(off, BLK), :] = o
        ones = jnp.ones((8, BLK), jnp.float32)
        s1 = jnp.dot(ones, o, preferred_element_type=jnp.float32,
                     precision=lax.Precision.HIGHEST)
        s2 = jnp.dot(ones, o * o, preferred_element_type=jnp.float32,
                     precision=lax.Precision.HIGHEST)

        @pl.when(pid == 0)
        def _():
            s1_acc[...] = s1
            s2_acc[...] = s2

        @pl.when(pid > 0)
        def _():
            s1_acc[...] += s1
            s2_acc[...] += s2

    @pl.when(pid >= NBLK)
    def _():
        off = pl.multiple_of((pid - NBLK) * BLK, BLK)
        o = o_sc[pl.ds(off, BLK), :]
        mean = s1_acc[0:1] * (1.0 / N)
        var = s2_acc[0:1] * (1.0 / N) - mean * mean
        inv = lax.rsqrt(var + BN_EPS)
        out = (o - mean) * inv * g_ref[...][None, :] + b_ref[...][None, :]
        if apply_act:
            out = _gelu(out)
            o_refs[0][...] = out[:, :DH]
            o_refs[1][...] = out[:, DH:]
        else:
            o_refs[0][...] = out


def _mlp(h0, h1, a0, a1, w1, w2, gamma, beta, apply_act):
    def in_map(pid):
        return (jnp.minimum(pid, NBLK - 1), 0)

    def out_map(pid):
        return (jnp.maximum(pid - NBLK, 0), 0)

    if apply_act:
        out_specs = [pl.BlockSpec((BLK, DH), out_map),
                     pl.BlockSpec((BLK, DH), out_map)]
        shapes = [jax.ShapeDtypeStruct((N, DH), jnp.float32),
                  jax.ShapeDtypeStruct((N, DH), jnp.float32)]
    else:
        out_specs = [pl.BlockSpec((BLK, D), out_map)]
        shapes = [jax.ShapeDtypeStruct((N, D), jnp.float32)]
    return pl.pallas_call(
        functools.partial(_mlp_body, apply_act),
        grid=(2 * NBLK,),
        in_specs=[pl.BlockSpec((BLK, DH), in_map),
                  pl.BlockSpec((BLK, DH), in_map),
                  pl.BlockSpec((BLK, DH), in_map),
                  pl.BlockSpec((BLK, DH), in_map),
                  pl.BlockSpec((D, 2 * D), lambda pid: (0, 0)),
                  pl.BlockSpec((2 * D, D), lambda pid: (0, 0)),
                  pl.BlockSpec((D,), lambda pid: (0,)),
                  pl.BlockSpec((D,), lambda pid: (0,))],
        out_specs=out_specs,
        out_shape=shapes,
        scratch_shapes=[pltpu.VMEM((N, D), jnp.float32),
                        pltpu.VMEM((8, D), jnp.float32),
                        pltpu.VMEM((8, D), jnp.float32)],
    )(h0, h1, a0, a1, w1, w2, gamma, beta)


def _seg_body(h0_hbm, h1_hbm, edge_hbm, out0_hbm, out1_hbm, *scr):
    sidx_v, didx_v = scr[0], scr[1]
    bufs = scr[2:2 + NBUF]
    acc_sh = scr[2 + NBUF]
    gsems = scr[3 + NBUF:3 + 2 * NBUF]
    ssems = scr[3 + 2 * NBUF:3 + 3 * NBUF]
    buf0 = bufs[0]
    c = lax.axis_index("c")
    s = lax.axis_index("s")

    pltpu.async_copy(edge_hbm.at[0, s], sidx_v, gsems[1])
    pltpu.async_copy(edge_hbm.at[1, s], didx_v, gsems[2])

    def zrow(i, _):
        def zcol(j, _):
            buf0[i, pl.ds(j * 16, 16)] = jnp.zeros((16,), jnp.float32)
            return 0
        return lax.fori_loop(0, DH // 16, zcol, 0)
    lax.fori_loop(0, ZR, zrow, 0)
    base = s * ROWS_PER_TILE
    nzk = jnp.where(s == NS - 1, ROWS_LAST // ZR, ROWS_PER_TILE // ZR)
    def zcopy(k, _):
        pltpu.async_copy(buf0.at[pl.ds(0, ZR)],
                         acc_sh.at[pl.ds(base + k * ZR, ZR)], gsems[0])
        return 0
    lax.fori_loop(0, nzk, zcopy, 0)
    def zdrain(k, _):
        pltpu.make_async_copy(buf0.at[pl.ds(0, ZR)],
                              acc_sh.at[pl.ds(base, ZR)], gsems[0]).wait()
        return 0
    lax.fori_loop(0, nzk, zdrain, 0)
    pltpu.make_async_copy(edge_hbm.at[0, s], sidx_v, gsems[1]).wait()
    pltpu.make_async_copy(edge_hbm.at[1, s], didx_v, gsems[2]).wait()
    plsc.subcore_barrier()

    def run(h_hbm):
        for b in range(NBUF):
            pltpu.async_copy(h_hbm.at[sidx_v.at[b]], bufs[b], gsems[b])

        def step(j, _):
            e0 = j * NBUF
            for b in range(NBUF):
                pltpu.make_async_copy(
                    h_hbm.at[sidx_v.at[e0 + b]], bufs[b], gsems[b]).wait()
                pltpu.async_copy(bufs[b], acc_sh.at[didx_v.at[e0 + b]],
                                 ssems[b], add=True)
            for b in range(NBUF):
                e2 = e0 + NBUF + b

                @pl.when(e2 < NCHUNK)
                def _():
                    pltpu.make_async_copy(
                        bufs[b], acc_sh.at[didx_v.at[e0 + b]],
                        ssems[b]).wait()
                    pltpu.async_copy(h_hbm.at[sidx_v.at[e2]], bufs[b],
                                     gsems[b])
            return 0
        lax.fori_loop(0, NCHUNK // NBUF, step, 0)
        for b in range(NBUF):
            pltpu.make_async_copy(
                bufs[b], acc_sh.at[didx_v.at[NCHUNK - NBUF + b]],
                ssems[b]).wait()

    @pl.when(c == 0)
    def _():
        run(h0_hbm)

    @pl.when(c == 1)
    def _():
        run(h1_hbm)

    plsc.subcore_barrier()

    def writeback(out_hbm):
        @pl.when(s < NS - 1)
        def _():
            pltpu.sync_copy(acc_sh.at[pl.ds(base, ROWS_PER_TILE)],
                            out_hbm.at[pl.ds(base, ROWS_PER_TILE)])

        @pl.when(s == NS - 1)
        def _():
            last = (NS - 1) * ROWS_PER_TILE
            pltpu.sync_copy(acc_sh.at[pl.ds(last, ROWS_LAST)],
                            out_hbm.at[pl.ds(last, ROWS_LAST)])

    @pl.when(c == 0)
    def _():
        writeback(out0_hbm)

    @pl.when(c == 1)
    def _():
        writeback(out1_hbm)


def _seg_sum(h0, h1, edge4):
    mesh = plsc.VectorSubcoreMesh(core_axis_name="c", subcore_axis_name="s")
    kern = pl.kernel(
        _seg_body,
        out_type=[jax.ShapeDtypeStruct((N, DH), jnp.float32),
                  jax.ShapeDtypeStruct((N, DH), jnp.float32)],
        mesh=mesh,
        compiler_params=pltpu.CompilerParams(use_tc_tiling_on_sc=False),
        scratch_types=(
            [pltpu.VMEM((NCHUNK, CHUNK), jnp.int32),
             pltpu.VMEM((NCHUNK, CHUNK), jnp.int32)]
            + [pltpu.VMEM((CHUNK, DH), jnp.float32)] * NBUF
            + [pltpu.VMEM_SHARED((N, DH), jnp.float32)]
            + [pltpu.SemaphoreType.DMA] * (2 * NBUF)
        ),
    )
    return kern(h0, h1, edge4)


def kernel(x, edge_index, W_enc, W1_0, W2_0, gamma_0, beta_0,
           W1_1, W2_1, gamma_1, beta_1):
    edge4 = edge_index.reshape(2, NS, NCHUNK, CHUNK)
    h0, h1 = _enc_matmul(x, W_enc)
    a0, a1 = _seg_sum(h0, h1, edge4)
    h0, h1 = _mlp(h0, h1, a0, a1, W1_0, W2_0, gamma_0, beta_0, True)
    a0, a1 = _seg_sum(h0, h1, edge4)
    (out,) = _mlp(h0, h1, a0, a1, W1_1, W2_1, gamma_1, beta_1, False)
    return out


# speedup vs baseline: 1.0659x; 1.0264x over previous
import functools

import jax
import jax.numpy as jnp
from jax import lax
from jax.experimental import pallas as pl
from jax.experimental.pallas import tpu as pltpu
from jax.experimental.pallas import tpu_sc as plsc

N = 10000
E = 320000
D = 128
DH = D // 2
BN_EPS = 1e-5

NC = 2
NS = 16
E_PER_T = E // NS
CHUNK = 125
NCHUNK = E_PER_T // CHUNK
NBUF = 5
ROWS_PER_TILE = 624
ROWS_LAST = N - (NS - 1) * ROWS_PER_TILE
ZR = 16


def _gelu(v):
    return v * 0.5 * (1.0 + lax.erf(v * 0.7071067811865476))


def _enc_body(x_ref, w_ref, o0_ref, o1_ref):
    h = jnp.dot(x_ref[...], w_ref[...], preferred_element_type=jnp.float32)
    o0_ref[...] = h[:, :DH]
    o1_ref[...] = h[:, DH:]


def _enc_matmul(x, w):
    return pl.pallas_call(
        _enc_body,
        out_shape=[jax.ShapeDtypeStruct((N, DH), jnp.float32),
                   jax.ShapeDtypeStruct((N, DH), jnp.float32)],
    )(x, w)


def _mlp_body(apply_act, h0_ref, h1_ref, a0_ref, a1_ref,
              w1_ref, w2_ref, g_ref, b_ref, *o_refs):
    m = jnp.concatenate(
        [h0_ref[...] + a0_ref[...], h1_ref[...] + a1_ref[...]], axis=1)
    hid = _gelu(jnp.dot(m, w1_ref[...], preferred_element_type=jnp.float32))
    o = jnp.dot(hid, w2_ref[...], preferred_element_type=jnp.float32)
    ones = jnp.ones((8, N), jnp.float32)
    s1 = jnp.dot(ones, o, preferred_element_type=jnp.float32,
                 precision=lax.Precision.HIGHEST)[0:1]
    s2 = jnp.dot(ones, o * o, preferred_element_type=jnp.float32,
                 precision=lax.Precision.HIGHEST)[0:1]
    mean = s1 * (1.0 / N)
    var = s2 * (1.0 / N) - mean * mean
    inv = lax.rsqrt(var + BN_EPS)
    out = (o - mean) * inv * g_ref[...][None, :] + b_ref[...][None, :]
    if apply_act:
        out = _gelu(out)
        o_refs[0][...] = out[:, :DH]
        o_refs[1][...] = out[:, DH:]
    else:
        o_refs[0][...] = out


def _mlp(h0, h1, a0, a1, w1, w2, gamma, beta, apply_act):
    if apply_act:
        shapes = [jax.ShapeDtypeStruct((N, DH), jnp.float32),
                  jax.ShapeDtypeStruct((N, DH), jnp.float32)]
    else:
        shapes = [jax.ShapeDtypeStruct((N, D), jnp.float32)]
    return pl.pallas_call(
        functools.partial(_mlp_body, apply_act),
        out_shape=shapes,
    )(h0, h1, a0, a1, w1, w2, gamma, beta)


def _seg_body(h0_hbm, h1_hbm, edge_hbm, out0_hbm, out1_hbm, *scr):
    sidx_v, didx_v = scr[0], scr[1]
    bufs = scr[2:2 + NBUF]
    acc_sh = scr[2 + NBUF]
    gsems = scr[3 + NBUF:3 + 2 * NBUF]
    ssems = scr[3 + 2 * NBUF:3 + 3 * NBUF]
    buf0 = bufs[0]
    c = lax.axis_index("c")
    s = lax.axis_index("s")

    pltpu.async_copy(edge_hbm.at[0, s], sidx_v, gsems[1])
    pltpu.async_copy(edge_hbm.at[1, s], didx_v, gsems[2])

    def zrow(i, _):
        def zcol(j, _):
            buf0[i, pl.ds(j * 16, 16)] = jnp.zeros((16,), jnp.float32)
            return 0
        return lax.fori_loop(0, DH // 16, zcol, 0)
    lax.fori_loop(0, ZR, zrow, 0)
    base = s * ROWS_PER_TILE
    nzk = jnp.where(s == NS - 1, ROWS_LAST // ZR, ROWS_PER_TILE // ZR)
    def zcopy(k, _):
        pltpu.async_copy(buf0.at[pl.ds(0, ZR)],
                         acc_sh.at[pl.ds(base + k * ZR, ZR)], gsems[0])
        return 0
    lax.fori_loop(0, nzk, zcopy, 0)
    def zdrain(k, _):
        pltpu.make_async_copy(buf0.at[pl.ds(0, ZR)],
                              acc_sh.at[pl.ds(base, ZR)], gsems[0]).wait()
        return 0
    lax.fori_loop(0, nzk, zdrain, 0)
    pltpu.make_async_copy(edge_hbm.at[0, s], sidx_v, gsems[1]).wait()
    pltpu.make_async_copy(edge_hbm.at[1, s], didx_v, gsems[2]).wait()
    plsc.subcore_barrier()

    def run(h_hbm):
        for b in range(NBUF):
            pltpu.async_copy(h_hbm.at[sidx_v.at[b]], bufs[b], gsems[b])

        def step(j, _):
            e0 = j * NBUF
            for b in range(NBUF):
                pltpu.make_async_copy(
                    h_hbm.at[sidx_v.at[e0 + b]], bufs[b], gsems[b]).wait()
                pltpu.async_copy(bufs[b], acc_sh.at[didx_v.at[e0 + b]],
                                 ssems[b], add=True)
            for b in range(NBUF):
                e2 = e0 + NBUF + b

                @pl.when(e2 < NCHUNK)
                def _():
                    pltpu.make_async_copy(
                        bufs[b], acc_sh.at[didx_v.at[e0 + b]],
                        ssems[b]).wait()
                    pltpu.async_copy(h_hbm.at[sidx_v.at[e2]], bufs[b],
                                     gsems[b])
            return 0
        lax.fori_loop(0, NCHUNK // NBUF, step, 0)
        for b in range(NBUF):
            pltpu.make_async_copy(
                bufs[b], acc_sh.at[didx_v.at[NCHUNK - NBUF + b]],
                ssems[b]).wait()

    @pl.when(c == 0)
    def _():
        run(h0_hbm)

    @pl.when(c == 1)
    def _():
        run(h1_hbm)

    plsc.subcore_barrier()

    def writeback(out_hbm):
        @pl.when(s < NS - 1)
        def _():
            pltpu.sync_copy(acc_sh.at[pl.ds(base, ROWS_PER_TILE)],
                            out_hbm.at[pl.ds(base, ROWS_PER_TILE)])

        @pl.when(s == NS - 1)
        def _():
            last = (NS - 1) * ROWS_PER_TILE
            pltpu.sync_copy(acc_sh.at[pl.ds(last, ROWS_LAST)],
                            out_hbm.at[pl.ds(last, ROWS_LAST)])

    @pl.when(c == 0)
    def _():
        writeback(out0_hbm)

    @pl.when(c == 1)
    def _():
        writeback(out1_hbm)


def _seg_sum(h0, h1, edge4):
    mesh = plsc.VectorSubcoreMesh(core_axis_name="c", subcore_axis_name="s")
    kern = pl.kernel(
        _seg_body,
        out_type=[jax.ShapeDtypeStruct((N, DH), jnp.float32),
                  jax.ShapeDtypeStruct((N, DH), jnp.float32)],
        mesh=mesh,
        compiler_params=pltpu.CompilerParams(use_tc_tiling_on_sc=False),
        scratch_types=(
            [pltpu.VMEM((NCHUNK, CHUNK), jnp.int32),
             pltpu.VMEM((NCHUNK, CHUNK), jnp.int32)]
            + [pltpu.VMEM((CHUNK, DH), jnp.float32)] * NBUF
            + [pltpu.VMEM_SHARED((N, DH), jnp.float32)]
            + [pltpu.SemaphoreType.DMA] * (2 * NBUF)
        ),
    )
    return kern(h0, h1, edge4)


def kernel(x, edge_index, W_enc, W1_0, W2_0, gamma_0, beta_0,
           W1_1, W2_1, gamma_1, beta_1):
    edge4 = edge_index.reshape(2, NS, NCHUNK, CHUNK)
    h0, h1 = _enc_matmul(x, W_enc)
    a0, a1 = _seg_sum(h0, h1, edge4)
    h0, h1 = _mlp(h0, h1, a0, a1, W1_0, W2_0, gamma_0, beta_0, True)
    a0, a1 = _seg_sum(h0, h1, edge4)
    (out,) = _mlp(h0, h1, a0, a1, W1_1, W2_1, gamma_1, beta_1, False)
    return out
